# baseline (speedup 1.0000x reference)
import functools

import jax
import jax.numpy as jnp
from jax import lax
from jax.experimental import pallas as pl
from jax.experimental.pallas import tpu as pltpu
from jax.experimental.pallas import tpu_sc as plsc

BB = 2
LL = 2048
DD = 768
II = 3072
EE = 8
NTOK = BB * LL
TILE = 256
NTILES = (2 * NTOK) // TILE + EE
PROWS = NTILES * TILE

NC = 2
NS = 16
NW = NC * NS
TW = NTOK // NW
CH = 64


def _cumsum_rows(a):
    n = a.shape[0]
    s = 1
    while s < n:
        a = a + jnp.concatenate([jnp.zeros((s, a.shape[1]), a.dtype), a[:-s, :]], axis=0)
        s *= 2
    return a


def _cumsum_lanes(a):
    s = 1
    while s < a.shape[1]:
        a = a + jnp.concatenate([jnp.zeros((1, s), a.dtype), a[:, :-s]], axis=1)
        s *= 2
    return a


def _router_body(x_ref, style_ref, wg_ref, pos0_ref, pos1_ref, w0_ref, w1_ref,
                 te_ref, usage_ref):
    neg_inf = jnp.float32(-jnp.inf)
    g = jnp.dot(x_ref[...], wg_ref[...], preferred_element_type=jnp.float32)
    lane = lax.broadcasted_iota(jnp.int32, (NTOK, EE), 1)
    v0 = jnp.max(g, axis=1, keepdims=True)
    e0 = jnp.min(jnp.where(g == v0, lane, EE), axis=1, keepdims=True)
    m = jnp.where(lane == e0, neg_inf, g)
    v1 = jnp.max(m, axis=1, keepdims=True)
    e1 = jnp.min(jnp.where(m == v1, lane, EE), axis=1, keepdims=True)
    ew = jnp.exp(v1 - v0)
    w0 = 1.0 / (1.0 + ew)
    w1 = ew / (1.0 + ew)

    oh0 = (lane == e0).astype(jnp.int32)
    oh1 = (lane == e1).astype(jnp.int32)
    c0 = _cumsum_rows(oh0)
    c1 = _cumsum_rows(oh1)
    tot0 = c0[NTOK - 1:NTOK, :]
    tot1 = c1[NTOK - 1:NTOK, :]
    counts = tot0 + tot1
    rank0 = jnp.sum(oh0 * (c0 - 1), axis=1, keepdims=True)
    rank1 = jnp.sum(oh1 * (tot0 + c1 - 1), axis=1, keepdims=True)

    pc = ((counts + (TILE - 1)) // TILE) * TILE
    off = _cumsum_lanes(pc) - pc
    pos0_ref[...] = jnp.sum(oh0 * off, axis=1, keepdims=True) + rank0
    pos1_ref[...] = jnp.sum(oh1 * off, axis=1, keepdims=True) + rank1
    w0_ref[...] = jnp.broadcast_to(w0, (NTOK, 128))
    w1_ref[...] = jnp.broadcast_to(w1, (NTOK, 128))

    endpos = off + pc
    tstart = lax.broadcasted_iota(jnp.int32, (NTILES, EE), 0) * TILE
    te = jnp.sum((tstart >= endpos).astype(jnp.int32), axis=1, keepdims=True)
    te_ref[...] = jnp.minimum(te, EE - 1)

    sg = jnp.dot(style_ref[...], wg_ref[...], preferred_element_type=jnp.float32)
    slane = lax.broadcasted_iota(jnp.int32, (8, EE), 1)
    sv0 = jnp.max(sg, axis=1, keepdims=True)
    se0 = jnp.min(jnp.where(sg == sv0, slane, EE), axis=1, keepdims=True)
    sm = jnp.where(slane == se0, neg_inf, sg)
    sv1 = jnp.max(sm, axis=1, keepdims=True)
    se1 = jnp.min(jnp.where(sm == sv1, slane, EE), axis=1, keepdims=True)
    rowvalid = lax.broadcasted_iota(jnp.int32, (8, EE), 0) < BB
    soh = ((slane == se0) | (slane == se1)) & rowvalid
    usage_ref[...] = counts.astype(jnp.float32) + LL * jnp.sum(
        soh.astype(jnp.float32), axis=0, keepdims=True)


def _router(x, style8, wg):
    return pl.pallas_call(
        _router_body,
        out_shape=(
            jax.ShapeDtypeStruct((NTOK, 1), jnp.int32),
            jax.ShapeDtypeStruct((NTOK, 1), jnp.int32),
            jax.ShapeDtypeStruct((NTOK, 128), jnp.float32),
            jax.ShapeDtypeStruct((NTOK, 128), jnp.float32),
            jax.ShapeDtypeStruct((NTILES, 1), jnp.int32),
            jax.ShapeDtypeStruct((1, EE), jnp.float32),
        ),
    )(x, style8, wg)


@functools.cache
def _dispatch_kernel():
    return functools.partial(
        pl.kernel,
        out_type=[
            jax.ShapeDtypeStruct((PROWS, DD), jnp.float32),
            jax.ShapeDtypeStruct((PROWS, 128), jnp.float32),
        ],
        mesh=plsc.VectorSubcoreMesh(core_axis_name="c", subcore_axis_name="s"),
        scratch_types=[
            pltpu.VMEM((CH, DD), jnp.float32),
            pltpu.VMEM((CH,), jnp.int32),
            pltpu.VMEM((CH,), jnp.int32),
            pltpu.VMEM((CH, 128), jnp.float32),
            pltpu.VMEM((CH, 128), jnp.float32),
            pltpu.SemaphoreType.DMA,
            pltpu.SemaphoreType.DMA,
        ],
    )(_dispatch_body)


def _dispatch_body(x_hbm, pos0_hbm, pos1_hbm, w0_hbm, w1_hbm, xs_hbm, ws_hbm,
                   buf, idx0, idx1, wb0, wb1, sem0, sem1):
    wid = lax.axis_index("s") * NC + lax.axis_index("c")
    base = wid * TW
    for c in range(TW // CH):
        cb = base + c * CH
        pltpu.sync_copy(x_hbm.at[pl.ds(cb, CH)], buf)
        pltpu.sync_copy(pos0_hbm.at[pl.ds(cb, CH)], idx0)
        pltpu.sync_copy(pos1_hbm.at[pl.ds(cb, CH)], idx1)
        pltpu.sync_copy(w0_hbm.at[pl.ds(cb, CH)], wb0)
        pltpu.sync_copy(w1_hbm.at[pl.ds(cb, CH)], wb1)
        c0 = pltpu.async_copy(buf, xs_hbm.at[idx0], sem0)
        c1 = pltpu.async_copy(buf, xs_hbm.at[idx1], sem1)
        c2 = pltpu.async_copy(wb0, ws_hbm.at[idx0], sem0)
        c3 = pltpu.async_copy(wb1, ws_hbm.at[idx1], sem1)
        c0.wait()
        c1.wait()
        c2.wait()
        c3.wait()


def _mm_body(te_ref, xs_ref, w1_ref, b1_ref, w2_ref, b2_ref, ws_ref, out_ref):
    h = jnp.dot(xs_ref[...].astype(jnp.bfloat16), w1_ref[0],
                preferred_element_type=jnp.float32)
    h = h + b1_ref[0]
    h = jax.nn.gelu(h, approximate=True)
    y = jnp.dot(h.astype(jnp.bfloat16), w2_ref[0],
                preferred_element_type=jnp.float32)
    y = y + b2_ref[0]
    out_ref[...] = y * ws_ref[:, 0:1]


def _grouped_mm(te, xs, w1, b1, w2, b2, ws):
    grid_spec = pltpu.PrefetchScalarGridSpec(
        num_scalar_prefetch=1,
        grid=(NTILES,),
        in_specs=[
            pl.BlockSpec((TILE, DD), lambda i, te: (i, 0)),
            pl.BlockSpec((1, DD, II), lambda i, te: (te[i], 0, 0)),
            pl.BlockSpec((1, 1, II), lambda i, te: (te[i], 0, 0)),
            pl.BlockSpec((1, II, DD), lambda i, te: (te[i], 0, 0)),
            pl.BlockSpec((1, 1, DD), lambda i, te: (te[i], 0, 0)),
            pl.BlockSpec((TILE, 128), lambda i, te: (i, 0)),
        ],
        out_specs=pl.BlockSpec((TILE, DD), lambda i, te: (i, 0)),
    )
    return pl.pallas_call(
        _mm_body,
        grid_spec=grid_spec,
        out_shape=jax.ShapeDtypeStruct((PROWS, DD), jnp.float32),
    )(te, xs, w1, b1, w2, b2, ws)


@functools.cache
def _combine_kernel():
    return functools.partial(
        pl.kernel,
        out_type=jax.ShapeDtypeStruct((NTOK, DD), jnp.float32),
        mesh=plsc.VectorSubcoreMesh(core_axis_name="c", subcore_axis_name="s"),
        scratch_types=[
            pltpu.VMEM((CH, DD), jnp.float32),
            pltpu.VMEM((CH, DD), jnp.float32),
            pltpu.VMEM((CH,), jnp.int32),
            pltpu.VMEM((CH,), jnp.int32),
            pltpu.SemaphoreType.DMA,
            pltpu.SemaphoreType.DMA,
        ],
    )(_combine_body)


def _combine_body(ys_hbm, pos0_hbm, pos1_hbm, out_hbm,
                  buf0, buf1, idx0, idx1, sem0, sem1):
    wid = lax.axis_index("s") * NC + lax.axis_index("c")
    base = wid * TW
    for c in range(TW // CH):
        cb = base + c * CH
        pltpu.sync_copy(pos0_hbm.at[pl.ds(cb, CH)], idx0)
        pltpu.sync_copy(pos1_hbm.at[pl.ds(cb, CH)], idx1)
        g0 = pltpu.async_copy(ys_hbm.at[idx0], buf0, sem0)
        g1 = pltpu.async_copy(ys_hbm.at[idx1], buf1, sem1)
        g0.wait()
        g1.wait()

        def row_body(i, carry):
            for j in range(DD // 16):
                plsc.addupdate(buf0.at[i, pl.ds(j * 16, 16)],
                               buf1[i, pl.ds(j * 16, 16)])
            return carry

        lax.fori_loop(0, CH, row_body, 0)
        pltpu.sync_copy(buf0, out_hbm.at[pl.ds(cb, CH)])


def kernel(hidden_states, style_emb, Wg, W1, b1, W2, b2):
    x = hidden_states.reshape(NTOK, DD)
    style8 = jnp.zeros((8, DD), jnp.float32).at[:BB].set(style_emb)
    pos0, pos1, w0b, w1b, te, usage = _router(x, style8, Wg)
    pos0 = pos0.reshape(NTOK)
    pos1 = pos1.reshape(NTOK)
    xs, ws = _dispatch_kernel()(x, pos0, pos1, w0b, w1b)
    ys = _grouped_mm(te.reshape(NTILES), xs, W1.astype(jnp.bfloat16),
                     b1.reshape(EE, 1, II), W2.astype(jnp.bfloat16),
                     b2.reshape(EE, 1, DD), ws)
    out = _combine_kernel()(ys, pos0, pos1)
    return out.reshape(BB, LL, DD), usage.reshape(EE)


# speedup vs baseline: 1.1926x; 1.1926x over previous
import functools

import jax
import jax.numpy as jnp
from jax import lax
from jax.experimental import pallas as pl
from jax.experimental.pallas import tpu as pltpu
from jax.experimental.pallas import tpu_sc as plsc

BB = 2
LL = 2048
DD = 768
II = 3072
EE = 8
NTOK = BB * LL
TILE = 256
NTILES = (2 * NTOK) // TILE + EE
PROWS = NTILES * TILE

NC = 2
NS = 16
NW = NC * NS
TW = NTOK // NW
CH = 64


def _cumsum_rows(a):
    n = a.shape[0]
    s = 1
    while s < n:
        a = a + jnp.concatenate([jnp.zeros((s, a.shape[1]), a.dtype), a[:-s, :]], axis=0)
        s *= 2
    return a


def _cumsum_lanes(a):
    s = 1
    while s < a.shape[1]:
        a = a + jnp.concatenate([jnp.zeros((1, s), a.dtype), a[:, :-s]], axis=1)
        s *= 2
    return a


def _router_body(x_ref, style_ref, wg_ref, pos0_ref, pos1_ref, w0_ref, w1_ref,
                 te_ref, usage_ref):
    neg_inf = jnp.float32(-jnp.inf)
    g = jnp.dot(x_ref[...], wg_ref[...], preferred_element_type=jnp.float32)
    lane = lax.broadcasted_iota(jnp.int32, (NTOK, EE), 1)
    v0 = jnp.max(g, axis=1, keepdims=True)
    e0 = jnp.min(jnp.where(g == v0, lane, EE), axis=1, keepdims=True)
    m = jnp.where(lane == e0, neg_inf, g)
    v1 = jnp.max(m, axis=1, keepdims=True)
    e1 = jnp.min(jnp.where(m == v1, lane, EE), axis=1, keepdims=True)
    ew = jnp.exp(v1 - v0)
    w0 = 1.0 / (1.0 + ew)
    w1 = ew / (1.0 + ew)

    oh0 = (lane == e0).astype(jnp.int32)
    oh1 = (lane == e1).astype(jnp.int32)
    c0 = _cumsum_rows(oh0)
    c1 = _cumsum_rows(oh1)
    tot0 = c0[NTOK - 1:NTOK, :]
    tot1 = c1[NTOK - 1:NTOK, :]
    counts = tot0 + tot1
    rank0 = jnp.sum(oh0 * (c0 - 1), axis=1, keepdims=True)
    rank1 = jnp.sum(oh1 * (tot0 + c1 - 1), axis=1, keepdims=True)

    pc = ((counts + (TILE - 1)) // TILE) * TILE
    off = _cumsum_lanes(pc) - pc
    pos0_ref[...] = jnp.sum(oh0 * off, axis=1, keepdims=True) + rank0
    pos1_ref[...] = jnp.sum(oh1 * off, axis=1, keepdims=True) + rank1
    w0_ref[...] = jnp.broadcast_to(w0, (NTOK, 16))
    w1_ref[...] = jnp.broadcast_to(w1, (NTOK, 16))

    endpos = off + pc
    tstart = lax.broadcasted_iota(jnp.int32, (NTILES, EE), 0) * TILE
    te = jnp.sum((tstart >= endpos).astype(jnp.int32), axis=1, keepdims=True)
    te_ref[...] = jnp.minimum(te, EE - 1)

    sg = jnp.dot(style_ref[...], wg_ref[...], preferred_element_type=jnp.float32)
    slane = lax.broadcasted_iota(jnp.int32, (8, EE), 1)
    sv0 = jnp.max(sg, axis=1, keepdims=True)
    se0 = jnp.min(jnp.where(sg == sv0, slane, EE), axis=1, keepdims=True)
    sm = jnp.where(slane == se0, neg_inf, sg)
    sv1 = jnp.max(sm, axis=1, keepdims=True)
    se1 = jnp.min(jnp.where(sm == sv1, slane, EE), axis=1, keepdims=True)
    rowvalid = lax.broadcasted_iota(jnp.int32, (8, EE), 0) < BB
    soh = ((slane == se0) | (slane == se1)) & rowvalid
    usage_ref[...] = counts.astype(jnp.float32) + LL * jnp.sum(
        soh.astype(jnp.float32), axis=0, keepdims=True)


def _router(x, style8, wg):
    return pl.pallas_call(
        _router_body,
        out_shape=(
            jax.ShapeDtypeStruct((NTOK, 1), jnp.int32),
            jax.ShapeDtypeStruct((NTOK, 1), jnp.int32),
            jax.ShapeDtypeStruct((NTOK, 16), jnp.float32),
            jax.ShapeDtypeStruct((NTOK, 16), jnp.float32),
            jax.ShapeDtypeStruct((NTILES, 1), jnp.int32),
            jax.ShapeDtypeStruct((1, EE), jnp.float32),
        ),
    )(x, style8, wg)


@functools.cache
def _dispatch_kernel():
    return functools.partial(
        pl.kernel,
        out_type=jax.ShapeDtypeStruct((PROWS, DD), jnp.float32),
        mesh=plsc.VectorSubcoreMesh(core_axis_name="c", subcore_axis_name="s"),
        scratch_types=[
            pltpu.VMEM((TW, DD), jnp.float32),
            pltpu.VMEM((TW,), jnp.int32),
            pltpu.VMEM((TW,), jnp.int32),
            pltpu.SemaphoreType.DMA,
            pltpu.SemaphoreType.DMA,
        ],
    )(_dispatch_body)


def _dispatch_body(x_hbm, pos0_hbm, pos1_hbm, xs_hbm,
                   buf, idx0, idx1, sem0, sem1):
    wid = lax.axis_index("s") * NC + lax.axis_index("c")
    base = wid * TW
    pltpu.sync_copy(x_hbm.at[pl.ds(base, TW)], buf)
    pltpu.sync_copy(pos0_hbm.at[pl.ds(base, TW)], idx0)
    pltpu.sync_copy(pos1_hbm.at[pl.ds(base, TW)], idx1)
    c0 = pltpu.async_copy(buf, xs_hbm.at[idx0], sem0)
    c1 = pltpu.async_copy(buf, xs_hbm.at[idx1], sem1)
    c0.wait()
    c1.wait()


def _mm_body(te_ref, xs_ref, w1_ref, b1_ref, w2_ref, b2_ref, out_ref):
    h = jnp.dot(xs_ref[...], w1_ref[0], preferred_element_type=jnp.float32)
    h = h + b1_ref[0]
    h = jax.nn.gelu(h, approximate=True)
    y = jnp.dot(h, w2_ref[0], preferred_element_type=jnp.float32)
    out_ref[...] = y + b2_ref[0]


def _grouped_mm(te, xs, w1, b1, w2, b2):
    grid_spec = pltpu.PrefetchScalarGridSpec(
        num_scalar_prefetch=1,
        grid=(NTILES,),
        in_specs=[
            pl.BlockSpec((TILE, DD), lambda i, te: (i, 0)),
            pl.BlockSpec((1, DD, II), lambda i, te: (te[i], 0, 0)),
            pl.BlockSpec((1, 1, II), lambda i, te: (te[i], 0, 0)),
            pl.BlockSpec((1, II, DD), lambda i, te: (te[i], 0, 0)),
            pl.BlockSpec((1, 1, DD), lambda i, te: (te[i], 0, 0)),
        ],
        out_specs=pl.BlockSpec((TILE, DD), lambda i, te: (i, 0)),
    )
    return pl.pallas_call(
        _mm_body,
        grid_spec=grid_spec,
        out_shape=jax.ShapeDtypeStruct((PROWS, DD), jnp.float32),
    )(te, xs, w1, b1, w2, b2)


@functools.cache
def _combine_kernel():
    return functools.partial(
        pl.kernel,
        out_type=jax.ShapeDtypeStruct((NTOK, DD), jnp.float32),
        mesh=plsc.VectorSubcoreMesh(core_axis_name="c", subcore_axis_name="s"),
        scratch_types=[
            pltpu.VMEM((CH, DD), jnp.float32),
            pltpu.VMEM((CH, DD), jnp.float32),
            pltpu.VMEM((CH,), jnp.int32),
            pltpu.VMEM((CH,), jnp.int32),
            pltpu.VMEM((CH, 16), jnp.float32),
            pltpu.VMEM((CH, 16), jnp.float32),
            pltpu.SemaphoreType.DMA,
            pltpu.SemaphoreType.DMA,
        ],
    )(_combine_body)


def _combine_body(ys_hbm, pos0_hbm, pos1_hbm, w0_hbm, w1_hbm, out_hbm,
                  buf0, buf1, idx0, idx1, w0v, w1v, sem0, sem1):
    wid = lax.axis_index("s") * NC + lax.axis_index("c")
    base = wid * TW
    for c in range(TW // CH):
        cb = base + c * CH
        pltpu.sync_copy(pos0_hbm.at[pl.ds(cb, CH)], idx0)
        pltpu.sync_copy(pos1_hbm.at[pl.ds(cb, CH)], idx1)
        pltpu.sync_copy(w0_hbm.at[pl.ds(cb, CH)], w0v)
        pltpu.sync_copy(w1_hbm.at[pl.ds(cb, CH)], w1v)
        g0 = pltpu.async_copy(ys_hbm.at[idx0], buf0, sem0)
        g1 = pltpu.async_copy(ys_hbm.at[idx1], buf1, sem1)
        g0.wait()
        g1.wait()

        def row_body(i, carry):
            w0row = w0v[i, :]
            w1row = w1v[i, :]
            for j in range(DD // 16):
                s = pl.ds(j * 16, 16)
                buf0[i, s] = w0row * buf0[i, s] + w1row * buf1[i, s]
            return carry

        lax.fori_loop(0, CH, row_body, 0)
        pltpu.sync_copy(buf0, out_hbm.at[pl.ds(cb, CH)])


def kernel(hidden_states, style_emb, Wg, W1, b1, W2, b2):
    x = hidden_states.reshape(NTOK, DD)
    style8 = jnp.zeros((8, DD), jnp.float32).at[:BB].set(style_emb)
    pos0, pos1, w0b, w1b, te, usage = _router(x, style8, Wg)
    pos0 = pos0.reshape(NTOK)
    pos1 = pos1.reshape(NTOK)
    xs = _dispatch_kernel()(x, pos0, pos1)
    ys = _grouped_mm(te.reshape(NTILES), xs, W1,
                     b1.reshape(EE, 1, II), W2, b2.reshape(EE, 1, DD))
    out = _combine_kernel()(ys, pos0, pos1, w0b, w1b)
    return out.reshape(BB, LL, DD), usage.reshape(EE)


# speedup vs baseline: 1.3008x; 1.0908x over previous
import functools

import jax
import jax.numpy as jnp
from jax import lax
from jax.experimental import pallas as pl
from jax.experimental.pallas import tpu as pltpu
from jax.experimental.pallas import tpu_sc as plsc

BB = 2
LL = 2048
DD = 768
II = 3072
EE = 8
NTOK = BB * LL
TILE = 256
NTILES = (2 * NTOK) // TILE + EE
PROWS = NTILES * TILE

NC = 2
NS = 16
NW = NC * NS
TW = NTOK // NW
CH = 64


def _cumsum_rows(a):
    n = a.shape[0]
    s = 1
    while s < n:
        a = a + jnp.concatenate([jnp.zeros((s, a.shape[1]), a.dtype), a[:-s, :]], axis=0)
        s *= 2
    return a


def _cumsum_lanes(a):
    s = 1
    while s < a.shape[1]:
        a = a + jnp.concatenate([jnp.zeros((1, s), a.dtype), a[:, :-s]], axis=1)
        s *= 2
    return a


def _router_body(x_ref, style_ref, wg_ref, pos0_ref, pos1_ref, w0_ref, w1_ref,
                 te_ref, usage_ref):
    neg_inf = jnp.float32(-jnp.inf)
    g = jnp.dot(x_ref[...], wg_ref[...], preferred_element_type=jnp.float32)
    lane = lax.broadcasted_iota(jnp.int32, (NTOK, EE), 1)
    v0 = jnp.max(g, axis=1, keepdims=True)
    e0 = jnp.min(jnp.where(g == v0, lane, EE), axis=1, keepdims=True)
    m = jnp.where(lane == e0, neg_inf, g)
    v1 = jnp.max(m, axis=1, keepdims=True)
    e1 = jnp.min(jnp.where(m == v1, lane, EE), axis=1, keepdims=True)
    ew = jnp.exp(v1 - v0)
    w0 = 1.0 / (1.0 + ew)
    w1 = ew / (1.0 + ew)

    oh0 = (lane == e0).astype(jnp.int32)
    oh1 = (lane == e1).astype(jnp.int32)
    c0 = _cumsum_rows(oh0)
    c1 = _cumsum_rows(oh1)
    tot0 = c0[NTOK - 1:NTOK, :]
    tot1 = c1[NTOK - 1:NTOK, :]
    counts = tot0 + tot1
    rank0 = jnp.sum(oh0 * (c0 - 1), axis=1, keepdims=True)
    rank1 = jnp.sum(oh1 * (tot0 + c1 - 1), axis=1, keepdims=True)

    pc = ((counts + (TILE - 1)) // TILE) * TILE
    off = _cumsum_lanes(pc) - pc
    pos0_ref[...] = jnp.sum(oh0 * off, axis=1, keepdims=True) + rank0
    pos1_ref[...] = jnp.sum(oh1 * off, axis=1, keepdims=True) + rank1
    w0_ref[...] = jnp.broadcast_to(w0, (NTOK, 16))
    w1_ref[...] = jnp.broadcast_to(w1, (NTOK, 16))

    endpos = off + pc
    tstart = lax.broadcasted_iota(jnp.int32, (NTILES, EE), 0) * TILE
    te = jnp.sum((tstart >= endpos).astype(jnp.int32), axis=1, keepdims=True)
    lane8 = lax.broadcasted_iota(jnp.int32, (1, EE), 1)
    maxe = jnp.max(jnp.where(counts > 0, lane8, 0), axis=1, keepdims=True)
    te = jnp.minimum(te, maxe)

    nt_i = lax.broadcasted_iota(jnp.int32, (NTILES, NTILES), 0)
    nt_j = lax.broadcasted_iota(jnp.int32, (NTILES, NTILES), 1)
    eye = (nt_i == nt_j).astype(jnp.float32)
    prev_te = jnp.concatenate(
        [jnp.full((1, 1), -1, jnp.int32), te[:-1, :]], axis=0)
    regstart = (te != prev_te).astype(jnp.int32)
    r = regstart
    s = 1
    while s < NTILES:
        r = r + jnp.concatenate(
            [jnp.zeros((s, 1), jnp.int32), r[:-s, :]], axis=0)
        s *= 2
    slot = (r - 1) % 2
    rs_row = lax.dot_general(regstart.astype(jnp.float32), eye,
                             (((0,), (0,)), ((), ())))
    te_row = lax.dot_general(te.astype(jnp.float32), eye,
                             (((0,), (0,)), ((), ())))
    cand = jnp.where((nt_j > nt_i) & (rs_row > 0), nt_j, NTILES)
    nexts = jnp.min(cand, axis=1, keepdims=True)
    hasnext = (nexts < NTILES).astype(jnp.int32)
    oh_next = (nt_j == nexts).astype(jnp.float32)
    nexte = lax.dot_general(
        oh_next, te_row, (((1,), (1,)), ((), ()))).astype(jnp.int32)
    meta = jnp.concatenate(
        [te, regstart, slot, nexte, hasnext,
         jnp.zeros((NTILES, 3), jnp.int32)], axis=1)
    te_ref[...] = meta

    sg = jnp.dot(style_ref[...], wg_ref[...], preferred_element_type=jnp.float32)
    slane = lax.broadcasted_iota(jnp.int32, (8, EE), 1)
    sv0 = jnp.max(sg, axis=1, keepdims=True)
    se0 = jnp.min(jnp.where(sg == sv0, slane, EE), axis=1, keepdims=True)
    sm = jnp.where(slane == se0, neg_inf, sg)
    sv1 = jnp.max(sm, axis=1, keepdims=True)
    se1 = jnp.min(jnp.where(sm == sv1, slane, EE), axis=1, keepdims=True)
    rowvalid = lax.broadcasted_iota(jnp.int32, (8, EE), 0) < BB
    soh = ((slane == se0) | (slane == se1)) & rowvalid
    usage_ref[...] = counts.astype(jnp.float32) + LL * jnp.sum(
        soh.astype(jnp.float32), axis=0, keepdims=True)


def _router(x, style8, wg):
    return pl.pallas_call(
        _router_body,
        out_shape=(
            jax.ShapeDtypeStruct((NTOK, 1), jnp.int32),
            jax.ShapeDtypeStruct((NTOK, 1), jnp.int32),
            jax.ShapeDtypeStruct((NTOK, 16), jnp.float32),
            jax.ShapeDtypeStruct((NTOK, 16), jnp.float32),
            jax.ShapeDtypeStruct((NTILES, 8), jnp.int32),
            jax.ShapeDtypeStruct((1, EE), jnp.float32),
        ),
    )(x, style8, wg)


@functools.cache
def _dispatch_kernel():
    return functools.partial(
        pl.kernel,
        out_type=jax.ShapeDtypeStruct((PROWS, DD), jnp.float32),
        mesh=plsc.VectorSubcoreMesh(core_axis_name="c", subcore_axis_name="s"),
        scratch_types=[
            pltpu.VMEM((TW, DD), jnp.float32),
            pltpu.VMEM((TW,), jnp.int32),
            pltpu.VMEM((TW,), jnp.int32),
            pltpu.SemaphoreType.DMA,
            pltpu.SemaphoreType.DMA,
        ],
    )(_dispatch_body)


def _dispatch_body(x_hbm, pos0_hbm, pos1_hbm, xs_hbm,
                   buf, idx0, idx1, sem0, sem1):
    wid = lax.axis_index("s") * NC + lax.axis_index("c")
    base = wid * TW
    pltpu.sync_copy(x_hbm.at[pl.ds(base, TW)], buf)
    pltpu.sync_copy(pos0_hbm.at[pl.ds(base, TW)], idx0)
    pltpu.sync_copy(pos1_hbm.at[pl.ds(base, TW)], idx1)
    c0 = pltpu.async_copy(buf, xs_hbm.at[idx0], sem0)
    c1 = pltpu.async_copy(buf, xs_hbm.at[idx1], sem1)
    c0.wait()
    c1.wait()


def _mm_body(meta_ref, xs_ref, w1_hbm, b1_ref, w2_hbm, b2_ref, out_ref,
             w1b0, w1b1, w2b0, w2b1, sem0, sem1):
    i = pl.program_id(0)
    te_i = meta_ref[i, 0]
    rs = meta_ref[i, 1]
    slot = meta_ref[i, 2]
    ne = meta_ref[i, 3]
    hn = meta_ref[i, 4]

    @pl.when(i == 0)
    def _():
        pltpu.make_async_copy(w1_hbm.at[te_i], w1b0, sem0).start()
        pltpu.make_async_copy(w2_hbm.at[te_i], w2b0, sem0).start()

    @pl.when(rs == 1)
    def _():
        @pl.when(slot == 0)
        def _():
            pltpu.make_async_copy(w1_hbm.at[te_i], w1b0, sem0).wait()
            pltpu.make_async_copy(w2_hbm.at[te_i], w2b0, sem0).wait()

            @pl.when(hn == 1)
            def _():
                pltpu.make_async_copy(w1_hbm.at[ne], w1b1, sem1).start()
                pltpu.make_async_copy(w2_hbm.at[ne], w2b1, sem1).start()

        @pl.when(slot == 1)
        def _():
            pltpu.make_async_copy(w1_hbm.at[te_i], w1b1, sem1).wait()
            pltpu.make_async_copy(w2_hbm.at[te_i], w2b1, sem1).wait()

            @pl.when(hn == 1)
            def _():
                pltpu.make_async_copy(w1_hbm.at[ne], w1b0, sem0).start()
                pltpu.make_async_copy(w2_hbm.at[ne], w2b0, sem0).start()

    def compute(w1ref, w2ref):
        h = jnp.dot(xs_ref[...], w1ref[...], preferred_element_type=jnp.float32)
        h = h + b1_ref[0]
        h = jax.nn.gelu(h, approximate=True)
        y = jnp.dot(h, w2ref[...], preferred_element_type=jnp.float32)
        out_ref[...] = y + b2_ref[0]

    @pl.when(slot == 0)
    def _():
        compute(w1b0, w2b0)

    @pl.when(slot == 1)
    def _():
        compute(w1b1, w2b1)


def _grouped_mm(meta, xs, w1, b1, w2, b2):
    grid_spec = pltpu.PrefetchScalarGridSpec(
        num_scalar_prefetch=1,
        grid=(NTILES,),
        in_specs=[
            pl.BlockSpec((TILE, DD), lambda i, m: (i, 0)),
            pl.BlockSpec(memory_space=pltpu.MemorySpace.HBM),
            pl.BlockSpec((1, 1, II), lambda i, m: (m[i, 0], 0, 0)),
            pl.BlockSpec(memory_space=pltpu.MemorySpace.HBM),
            pl.BlockSpec((1, 1, DD), lambda i, m: (m[i, 0], 0, 0)),
        ],
        out_specs=pl.BlockSpec((TILE, DD), lambda i, m: (i, 0)),
        scratch_shapes=[
            pltpu.VMEM((DD, II), jnp.float32),
            pltpu.VMEM((DD, II), jnp.float32),
            pltpu.VMEM((II, DD), jnp.float32),
            pltpu.VMEM((II, DD), jnp.float32),
            pltpu.SemaphoreType.DMA,
            pltpu.SemaphoreType.DMA,
        ],
    )
    return pl.pallas_call(
        _mm_body,
        grid_spec=grid_spec,
        out_shape=jax.ShapeDtypeStruct((PROWS, DD), jnp.float32),
    )(meta, xs, w1, b1, w2, b2)


@functools.cache
def _combine_kernel():
    return functools.partial(
        pl.kernel,
        out_type=jax.ShapeDtypeStruct((NTOK, DD), jnp.float32),
        mesh=plsc.VectorSubcoreMesh(core_axis_name="c", subcore_axis_name="s"),
        scratch_types=[
            pltpu.VMEM((CH, DD), jnp.float32),
            pltpu.VMEM((CH, DD), jnp.float32),
            pltpu.VMEM((CH,), jnp.int32),
            pltpu.VMEM((CH,), jnp.int32),
            pltpu.VMEM((CH, 16), jnp.float32),
            pltpu.VMEM((CH, 16), jnp.float32),
            pltpu.SemaphoreType.DMA,
            pltpu.SemaphoreType.DMA,
        ],
    )(_combine_body)


def _combine_body(ys_hbm, pos0_hbm, pos1_hbm, w0_hbm, w1_hbm, out_hbm,
                  buf0, buf1, idx0, idx1, w0v, w1v, sem0, sem1):
    wid = lax.axis_index("s") * NC + lax.axis_index("c")
    base = wid * TW
    for c in range(TW // CH):
        cb = base + c * CH
        pltpu.sync_copy(pos0_hbm.at[pl.ds(cb, CH)], idx0)
        pltpu.sync_copy(pos1_hbm.at[pl.ds(cb, CH)], idx1)
        pltpu.sync_copy(w0_hbm.at[pl.ds(cb, CH)], w0v)
        pltpu.sync_copy(w1_hbm.at[pl.ds(cb, CH)], w1v)
        g0 = pltpu.async_copy(ys_hbm.at[idx0], buf0, sem0)
        g1 = pltpu.async_copy(ys_hbm.at[idx1], buf1, sem1)
        g0.wait()
        g1.wait()

        def row_body(i, carry):
            w0row = w0v[i, :]
            w1row = w1v[i, :]
            for j in range(DD // 16):
                s = pl.ds(j * 16, 16)
                buf0[i, s] = w0row * buf0[i, s] + w1row * buf1[i, s]
            return carry

        lax.fori_loop(0, CH, row_body, 0)
        pltpu.sync_copy(buf0, out_hbm.at[pl.ds(cb, CH)])


def kernel(hidden_states, style_emb, Wg, W1, b1, W2, b2):
    x = hidden_states.reshape(NTOK, DD)
    style8 = jnp.zeros((8, DD), jnp.float32).at[:BB].set(style_emb)
    pos0, pos1, w0b, w1b, meta, usage = _router(x, style8, Wg)
    pos0 = pos0.reshape(NTOK)
    pos1 = pos1.reshape(NTOK)
    xs = _dispatch_kernel()(x, pos0, pos1)
    ys = _grouped_mm(meta, xs, W1,
                     b1.reshape(EE, 1, II), W2, b2.reshape(EE, 1, DD))
    out = _combine_kernel()(ys, pos0, pos1, w0b, w1b)
    return out.reshape(BB, LL, DD), usage.reshape(EE)


# speedup vs baseline: 1.3706x; 1.0536x over previous
import functools

import jax
import jax.numpy as jnp
from jax import lax
from jax.experimental import pallas as pl
from jax.experimental.pallas import tpu as pltpu
from jax.experimental.pallas import tpu_sc as plsc

BB = 2
LL = 2048
DD = 768
II = 3072
EE = 8
NTOK = BB * LL
TILE = 256
NTILES = (2 * NTOK) // TILE + EE
PROWS = NTILES * TILE

NC = 2
NS = 16
NW = NC * NS
TW = NTOK // NW
CH = 64


def _cumsum_rows(a):
    n = a.shape[0]
    s = 1
    while s < n:
        a = a + jnp.concatenate([jnp.zeros((s, a.shape[1]), a.dtype), a[:-s, :]], axis=0)
        s *= 2
    return a


def _cumsum_lanes(a):
    s = 1
    while s < a.shape[1]:
        a = a + jnp.concatenate([jnp.zeros((1, s), a.dtype), a[:, :-s]], axis=1)
        s *= 2
    return a


def _router_body(x_ref, style_ref, wg_ref, pos0_ref, pos1_ref, w0_ref, w1_ref,
                 te_ref, usage_ref):
    neg_inf = jnp.float32(-jnp.inf)
    g = jnp.dot(x_ref[...], wg_ref[...], preferred_element_type=jnp.float32)
    lane = lax.broadcasted_iota(jnp.int32, (NTOK, EE), 1)
    v0 = jnp.max(g, axis=1, keepdims=True)
    e0 = jnp.min(jnp.where(g == v0, lane, EE), axis=1, keepdims=True)
    m = jnp.where(lane == e0, neg_inf, g)
    v1 = jnp.max(m, axis=1, keepdims=True)
    e1 = jnp.min(jnp.where(m == v1, lane, EE), axis=1, keepdims=True)
    ew = jnp.exp(v1 - v0)
    w0 = 1.0 / (1.0 + ew)
    w1 = ew / (1.0 + ew)

    oh0 = (lane == e0).astype(jnp.int32)
    oh1 = (lane == e1).astype(jnp.int32)
    c0 = _cumsum_rows(oh0)
    c1 = _cumsum_rows(oh1)
    tot0 = c0[NTOK - 1:NTOK, :]
    tot1 = c1[NTOK - 1:NTOK, :]
    counts = tot0 + tot1
    rank0 = jnp.sum(oh0 * (c0 - 1), axis=1, keepdims=True)
    rank1 = jnp.sum(oh1 * (tot0 + c1 - 1), axis=1, keepdims=True)

    pc = ((counts + (TILE - 1)) // TILE) * TILE
    off = _cumsum_lanes(pc) - pc
    pos0_ref[...] = jnp.sum(oh0 * off, axis=1, keepdims=True) + rank0
    pos1_ref[...] = jnp.sum(oh1 * off, axis=1, keepdims=True) + rank1
    w0_ref[...] = jnp.broadcast_to(w0, (NTOK, 16))
    w1_ref[...] = jnp.broadcast_to(w1, (NTOK, 16))

    endpos = off + pc
    tstart = lax.broadcasted_iota(jnp.int32, (NTILES, EE), 0) * TILE
    te = jnp.sum((tstart >= endpos).astype(jnp.int32), axis=1, keepdims=True)
    lane8 = lax.broadcasted_iota(jnp.int32, (1, EE), 1)
    maxe = jnp.max(jnp.where(counts > 0, lane8, 0), axis=1, keepdims=True)
    te = jnp.minimum(te, maxe)

    nt_i = lax.broadcasted_iota(jnp.int32, (NTILES, NTILES), 0)
    nt_j = lax.broadcasted_iota(jnp.int32, (NTILES, NTILES), 1)
    eye = (nt_i == nt_j).astype(jnp.float32)
    prev_te = jnp.concatenate(
        [jnp.full((1, 1), -1, jnp.int32), te[:-1, :]], axis=0)
    regstart = (te != prev_te).astype(jnp.int32)
    r = regstart
    s = 1
    while s < NTILES:
        r = r + jnp.concatenate(
            [jnp.zeros((s, 1), jnp.int32), r[:-s, :]], axis=0)
        s *= 2
    slot = (r - 1) % 2
    rs_row = lax.dot_general(regstart.astype(jnp.float32), eye,
                             (((0,), (0,)), ((), ())))
    te_row = lax.dot_general(te.astype(jnp.float32), eye,
                             (((0,), (0,)), ((), ())))
    cand = jnp.where((nt_j > nt_i) & (rs_row > 0), nt_j, NTILES)
    nexts = jnp.min(cand, axis=1, keepdims=True)
    hasnext = (nexts < NTILES).astype(jnp.int32)
    oh_next = (nt_j == nexts).astype(jnp.float32)
    nexte = lax.dot_general(
        oh_next, te_row, (((1,), (1,)), ((), ()))).astype(jnp.int32)
    used = jnp.sum(pc, axis=1, keepdims=True)
    tvalid = (lax.broadcasted_iota(jnp.int32, (NTILES, 1), 0) * TILE
              < used).astype(jnp.int32)
    meta = jnp.concatenate(
        [te, regstart, slot, nexte, hasnext, tvalid,
         jnp.zeros((NTILES, 2), jnp.int32)], axis=1)
    te_ref[...] = meta

    sg = jnp.dot(style_ref[...], wg_ref[...], preferred_element_type=jnp.float32)
    slane = lax.broadcasted_iota(jnp.int32, (8, EE), 1)
    sv0 = jnp.max(sg, axis=1, keepdims=True)
    se0 = jnp.min(jnp.where(sg == sv0, slane, EE), axis=1, keepdims=True)
    sm = jnp.where(slane == se0, neg_inf, sg)
    sv1 = jnp.max(sm, axis=1, keepdims=True)
    se1 = jnp.min(jnp.where(sm == sv1, slane, EE), axis=1, keepdims=True)
    rowvalid = lax.broadcasted_iota(jnp.int32, (8, EE), 0) < BB
    soh = ((slane == se0) | (slane == se1)) & rowvalid
    usage_ref[...] = counts.astype(jnp.float32) + LL * jnp.sum(
        soh.astype(jnp.float32), axis=0, keepdims=True)


def _router(x, style8, wg):
    return pl.pallas_call(
        _router_body,
        out_shape=(
            jax.ShapeDtypeStruct((NTOK, 1), jnp.int32),
            jax.ShapeDtypeStruct((NTOK, 1), jnp.int32),
            jax.ShapeDtypeStruct((NTOK, 16), jnp.float32),
            jax.ShapeDtypeStruct((NTOK, 16), jnp.float32),
            jax.ShapeDtypeStruct((NTILES, 8), jnp.int32),
            jax.ShapeDtypeStruct((1, EE), jnp.float32),
        ),
    )(x, style8, wg)


@functools.cache
def _dispatch_kernel():
    return functools.partial(
        pl.kernel,
        out_type=jax.ShapeDtypeStruct((PROWS, DD), jnp.float32),
        mesh=plsc.VectorSubcoreMesh(core_axis_name="c", subcore_axis_name="s"),
        scratch_types=[
            pltpu.VMEM((TW, DD), jnp.float32),
            pltpu.VMEM((TW,), jnp.int32),
            pltpu.VMEM((TW,), jnp.int32),
            pltpu.SemaphoreType.DMA,
            pltpu.SemaphoreType.DMA,
        ],
    )(_dispatch_body)


def _dispatch_body(x_hbm, pos0_hbm, pos1_hbm, xs_hbm,
                   buf, idx0, idx1, sem0, sem1):
    wid = lax.axis_index("s") * NC + lax.axis_index("c")
    base = wid * TW
    pltpu.sync_copy(x_hbm.at[pl.ds(base, TW)], buf)
    pltpu.sync_copy(pos0_hbm.at[pl.ds(base, TW)], idx0)
    pltpu.sync_copy(pos1_hbm.at[pl.ds(base, TW)], idx1)
    c0 = pltpu.async_copy(buf, xs_hbm.at[idx0], sem0)
    c1 = pltpu.async_copy(buf, xs_hbm.at[idx1], sem1)
    c0.wait()
    c1.wait()


def _mm_body(meta_ref, xs_ref, w1_hbm, b1_ref, w2_hbm, b2_ref, out_ref,
             w1b0, w1b1, w2b0, w2b1, sem0, sem1):
    i = pl.program_id(0)
    te_i = meta_ref[i, 0]
    rs = meta_ref[i, 1]
    slot = meta_ref[i, 2]
    ne = meta_ref[i, 3]
    hn = meta_ref[i, 4]

    @pl.when(i == 0)
    def _():
        pltpu.make_async_copy(w1_hbm.at[te_i], w1b0, sem0).start()
        pltpu.make_async_copy(w2_hbm.at[te_i], w2b0, sem0).start()

    @pl.when(rs == 1)
    def _():
        @pl.when(slot == 0)
        def _():
            pltpu.make_async_copy(w1_hbm.at[te_i], w1b0, sem0).wait()
            pltpu.make_async_copy(w2_hbm.at[te_i], w2b0, sem0).wait()

            @pl.when(hn == 1)
            def _():
                pltpu.make_async_copy(w1_hbm.at[ne], w1b1, sem1).start()
                pltpu.make_async_copy(w2_hbm.at[ne], w2b1, sem1).start()

        @pl.when(slot == 1)
        def _():
            pltpu.make_async_copy(w1_hbm.at[te_i], w1b1, sem1).wait()
            pltpu.make_async_copy(w2_hbm.at[te_i], w2b1, sem1).wait()

            @pl.when(hn == 1)
            def _():
                pltpu.make_async_copy(w1_hbm.at[ne], w1b0, sem0).start()
                pltpu.make_async_copy(w2_hbm.at[ne], w2b0, sem0).start()

    tv = meta_ref[i, 5]

    def compute(w1ref, w2ref):
        h = jnp.dot(xs_ref[...], w1ref[...], preferred_element_type=jnp.float32)
        h = h + b1_ref[0]
        h = jax.nn.gelu(h, approximate=True)
        y = jnp.dot(h, w2ref[...], preferred_element_type=jnp.float32)
        out_ref[...] = y + b2_ref[0]

    @pl.when((slot == 0) & (tv == 1))
    def _():
        compute(w1b0, w2b0)

    @pl.when((slot == 1) & (tv == 1))
    def _():
        compute(w1b1, w2b1)


def _grouped_mm(meta, xs, w1, b1, w2, b2):
    grid_spec = pltpu.PrefetchScalarGridSpec(
        num_scalar_prefetch=1,
        grid=(NTILES,),
        in_specs=[
            pl.BlockSpec((TILE, DD), lambda i, m: (i, 0)),
            pl.BlockSpec(memory_space=pltpu.MemorySpace.HBM),
            pl.BlockSpec((1, 1, II), lambda i, m: (m[i, 0], 0, 0)),
            pl.BlockSpec(memory_space=pltpu.MemorySpace.HBM),
            pl.BlockSpec((1, 1, DD), lambda i, m: (m[i, 0], 0, 0)),
        ],
        out_specs=pl.BlockSpec((TILE, DD), lambda i, m: (i, 0)),
        scratch_shapes=[
            pltpu.VMEM((DD, II), jnp.float32),
            pltpu.VMEM((DD, II), jnp.float32),
            pltpu.VMEM((II, DD), jnp.float32),
            pltpu.VMEM((II, DD), jnp.float32),
            pltpu.SemaphoreType.DMA,
            pltpu.SemaphoreType.DMA,
        ],
    )
    return pl.pallas_call(
        _mm_body,
        grid_spec=grid_spec,
        out_shape=jax.ShapeDtypeStruct((PROWS, DD), jnp.float32),
    )(meta, xs, w1, b1, w2, b2)


@functools.cache
def _combine_kernel():
    return functools.partial(
        pl.kernel,
        out_type=jax.ShapeDtypeStruct((NTOK, DD), jnp.float32),
        mesh=plsc.VectorSubcoreMesh(core_axis_name="c", subcore_axis_name="s"),
        scratch_types=[
            pltpu.VMEM((CH, DD), jnp.float32),
            pltpu.VMEM((CH, DD), jnp.float32),
            pltpu.VMEM((CH,), jnp.int32),
            pltpu.VMEM((CH,), jnp.int32),
            pltpu.VMEM((CH, 16), jnp.float32),
            pltpu.VMEM((CH, 16), jnp.float32),
            pltpu.SemaphoreType.DMA,
            pltpu.SemaphoreType.DMA,
        ],
    )(_combine_body)


def _combine_body(ys_hbm, pos0_hbm, pos1_hbm, w0_hbm, w1_hbm, out_hbm,
                  buf0, buf1, idx0, idx1, w0v, w1v, sem0, sem1):
    wid = lax.axis_index("s") * NC + lax.axis_index("c")
    base = wid * TW
    for c in range(TW // CH):
        cb = base + c * CH
        pltpu.sync_copy(pos0_hbm.at[pl.ds(cb, CH)], idx0)
        pltpu.sync_copy(pos1_hbm.at[pl.ds(cb, CH)], idx1)
        pltpu.sync_copy(w0_hbm.at[pl.ds(cb, CH)], w0v)
        pltpu.sync_copy(w1_hbm.at[pl.ds(cb, CH)], w1v)
        g0 = pltpu.async_copy(ys_hbm.at[idx0], buf0, sem0)
        g1 = pltpu.async_copy(ys_hbm.at[idx1], buf1, sem1)
        g0.wait()
        g1.wait()

        def row_body(i, carry):
            w0row = w0v[i, :]
            w1row = w1v[i, :]
            for j in range(DD // 16):
                s = pl.ds(j * 16, 16)
                buf0[i, s] = w0row * buf0[i, s] + w1row * buf1[i, s]
            return carry

        lax.fori_loop(0, CH, row_body, 0)
        pltpu.sync_copy(buf0, out_hbm.at[pl.ds(cb, CH)])


def kernel(hidden_states, style_emb, Wg, W1, b1, W2, b2):
    x = hidden_states.reshape(NTOK, DD)
    style8 = jnp.zeros((8, DD), jnp.float32).at[:BB].set(style_emb)
    pos0, pos1, w0b, w1b, meta, usage = _router(x, style8, Wg)
    pos0 = pos0.reshape(NTOK)
    pos1 = pos1.reshape(NTOK)
    xs = _dispatch_kernel()(x, pos0, pos1)
    ys = _grouped_mm(meta, xs, W1,
                     b1.reshape(EE, 1, II), W2, b2.reshape(EE, 1, DD))
    out = _combine_kernel()(ys, pos0, pos1, w0b, w1b)
    return out.reshape(BB, LL, DD), usage.reshape(EE)


# speedup vs baseline: 1.3915x; 1.0153x over previous
import functools

import jax
import jax.numpy as jnp
from jax import lax
from jax.experimental import pallas as pl
from jax.experimental.pallas import tpu as pltpu
from jax.experimental.pallas import tpu_sc as plsc

BB = 2
LL = 2048
DD = 768
II = 3072
EE = 8
NTOK = BB * LL
TILE = 256
NTILES = (2 * NTOK) // TILE + EE
PROWS = NTILES * TILE

NC = 2
NS = 16
NW = NC * NS
TW = NTOK // NW
CH = 64


def _cumsum_rows(a):
    n = a.shape[0]
    s = 1
    while s < n:
        a = a + jnp.concatenate([jnp.zeros((s, a.shape[1]), a.dtype), a[:-s, :]], axis=0)
        s *= 2
    return a


def _cumsum_lanes(a):
    s = 1
    while s < a.shape[1]:
        a = a + jnp.concatenate([jnp.zeros((1, s), a.dtype), a[:, :-s]], axis=1)
        s *= 2
    return a


def _router_body(x_ref, style_ref, wg_ref, pos0_ref, pos1_ref, w0_ref, w1_ref,
                 te_ref, usage_ref):
    neg_inf = jnp.float32(-jnp.inf)
    g = jnp.dot(x_ref[...], wg_ref[...], preferred_element_type=jnp.float32)
    lane = lax.broadcasted_iota(jnp.int32, (NTOK, EE), 1)
    v0 = jnp.max(g, axis=1, keepdims=True)
    e0 = jnp.min(jnp.where(g == v0, lane, EE), axis=1, keepdims=True)
    m = jnp.where(lane == e0, neg_inf, g)
    v1 = jnp.max(m, axis=1, keepdims=True)
    e1 = jnp.min(jnp.where(m == v1, lane, EE), axis=1, keepdims=True)
    ew = jnp.exp(v1 - v0)
    w0 = 1.0 / (1.0 + ew)
    w1 = ew / (1.0 + ew)

    oh0 = (lane == e0).astype(jnp.int32)
    oh1 = (lane == e1).astype(jnp.int32)
    c0 = _cumsum_rows(oh0)
    c1 = _cumsum_rows(oh1)
    tot0 = c0[NTOK - 1:NTOK, :]
    tot1 = c1[NTOK - 1:NTOK, :]
    counts = tot0 + tot1
    rank0 = jnp.sum(oh0 * (c0 - 1), axis=1, keepdims=True)
    rank1 = jnp.sum(oh1 * (tot0 + c1 - 1), axis=1, keepdims=True)

    pc = ((counts + (TILE - 1)) // TILE) * TILE
    off = _cumsum_lanes(pc) - pc
    pos0_ref[...] = jnp.sum(oh0 * off, axis=1, keepdims=True) + rank0
    pos1_ref[...] = jnp.sum(oh1 * off, axis=1, keepdims=True) + rank1
    w0_ref[...] = jnp.broadcast_to(w0, (NTOK, 16))
    w1_ref[...] = jnp.broadcast_to(w1, (NTOK, 16))

    endpos = off + pc
    tstart = lax.broadcasted_iota(jnp.int32, (NTILES, EE), 0) * TILE
    te = jnp.sum((tstart >= endpos).astype(jnp.int32), axis=1, keepdims=True)
    lane8 = lax.broadcasted_iota(jnp.int32, (1, EE), 1)
    maxe = jnp.max(jnp.where(counts > 0, lane8, 0), axis=1, keepdims=True)
    te = jnp.minimum(te, maxe)

    nt_i = lax.broadcasted_iota(jnp.int32, (NTILES, NTILES), 0)
    nt_j = lax.broadcasted_iota(jnp.int32, (NTILES, NTILES), 1)
    eye = (nt_i == nt_j).astype(jnp.float32)
    prev_te = jnp.concatenate(
        [jnp.full((1, 1), -1, jnp.int32), te[:-1, :]], axis=0)
    regstart = (te != prev_te).astype(jnp.int32)
    r = regstart
    s = 1
    while s < NTILES:
        r = r + jnp.concatenate(
            [jnp.zeros((s, 1), jnp.int32), r[:-s, :]], axis=0)
        s *= 2
    slot = (r - 1) % 2
    rs_row = lax.dot_general(regstart.astype(jnp.float32), eye,
                             (((0,), (0,)), ((), ())))
    te_row = lax.dot_general(te.astype(jnp.float32), eye,
                             (((0,), (0,)), ((), ())))
    cand = jnp.where((nt_j > nt_i) & (rs_row > 0), nt_j, NTILES)
    nexts = jnp.min(cand, axis=1, keepdims=True)
    hasnext = (nexts < NTILES).astype(jnp.int32)
    oh_next = (nt_j == nexts).astype(jnp.float32)
    nexte = lax.dot_general(
        oh_next, te_row, (((1,), (1,)), ((), ()))).astype(jnp.int32)
    used = jnp.sum(pc, axis=1, keepdims=True)
    tvalid = (lax.broadcasted_iota(jnp.int32, (NTILES, 1), 0) * TILE
              < used).astype(jnp.int32)
    meta = jnp.concatenate(
        [te, regstart, slot, nexte, hasnext, tvalid,
         jnp.zeros((NTILES, 2), jnp.int32)], axis=1)
    te_ref[...] = meta

    sg = jnp.dot(style_ref[...], wg_ref[...], preferred_element_type=jnp.float32)
    slane = lax.broadcasted_iota(jnp.int32, (8, EE), 1)
    sv0 = jnp.max(sg, axis=1, keepdims=True)
    se0 = jnp.min(jnp.where(sg == sv0, slane, EE), axis=1, keepdims=True)
    sm = jnp.where(slane == se0, neg_inf, sg)
    sv1 = jnp.max(sm, axis=1, keepdims=True)
    se1 = jnp.min(jnp.where(sm == sv1, slane, EE), axis=1, keepdims=True)
    rowvalid = lax.broadcasted_iota(jnp.int32, (8, EE), 0) < BB
    soh = ((slane == se0) | (slane == se1)) & rowvalid
    usage_ref[...] = counts.astype(jnp.float32) + LL * jnp.sum(
        soh.astype(jnp.float32), axis=0, keepdims=True)


def _router(x, style8, wg):
    return pl.pallas_call(
        _router_body,
        out_shape=(
            jax.ShapeDtypeStruct((NTOK, 1), jnp.int32),
            jax.ShapeDtypeStruct((NTOK, 1), jnp.int32),
            jax.ShapeDtypeStruct((NTOK, 16), jnp.float32),
            jax.ShapeDtypeStruct((NTOK, 16), jnp.float32),
            jax.ShapeDtypeStruct((NTILES, 8), jnp.int32),
            jax.ShapeDtypeStruct((1, EE), jnp.float32),
        ),
    )(x, style8, wg)


@functools.cache
def _dispatch_kernel():
    return functools.partial(
        pl.kernel,
        out_type=jax.ShapeDtypeStruct((PROWS, DD), jnp.float32),
        mesh=plsc.VectorSubcoreMesh(core_axis_name="c", subcore_axis_name="s"),
        scratch_types=[
            pltpu.VMEM((TW, DD), jnp.float32),
            pltpu.VMEM((TW,), jnp.int32),
            pltpu.VMEM((TW,), jnp.int32),
            pltpu.SemaphoreType.DMA,
            pltpu.SemaphoreType.DMA,
        ],
    )(_dispatch_body)


def _dispatch_body(x_hbm, pos0_hbm, pos1_hbm, xs_hbm,
                   buf, idx0, idx1, sem0, sem1):
    wid = lax.axis_index("s") * NC + lax.axis_index("c")
    base = wid * TW
    pltpu.sync_copy(x_hbm.at[pl.ds(base, TW)], buf)
    pltpu.sync_copy(pos0_hbm.at[pl.ds(base, TW)], idx0)
    pltpu.sync_copy(pos1_hbm.at[pl.ds(base, TW)], idx1)
    c0 = pltpu.async_copy(buf, xs_hbm.at[idx0], sem0)
    c1 = pltpu.async_copy(buf, xs_hbm.at[idx1], sem1)
    c0.wait()
    c1.wait()


def _mm_body(meta_ref, xs_ref, w1_hbm, b1_ref, w2_hbm, b2_ref, out_ref,
             w1b0, w1b1, w2b0, w2b1, sem0, sem1):
    i = pl.program_id(0)
    te_i = meta_ref[i, 0]
    rs = meta_ref[i, 1]
    slot = meta_ref[i, 2]
    ne = meta_ref[i, 3]
    hn = meta_ref[i, 4]

    @pl.when(i == 0)
    def _():
        pltpu.make_async_copy(w1_hbm.at[te_i], w1b0, sem0).start()
        pltpu.make_async_copy(w2_hbm.at[te_i], w2b0, sem0).start()

    @pl.when(rs == 1)
    def _():
        @pl.when(slot == 0)
        def _():
            pltpu.make_async_copy(w1_hbm.at[te_i], w1b0, sem0).wait()
            pltpu.make_async_copy(w2_hbm.at[te_i], w2b0, sem0).wait()

            @pl.when(hn == 1)
            def _():
                pltpu.make_async_copy(w1_hbm.at[ne], w1b1, sem1).start()
                pltpu.make_async_copy(w2_hbm.at[ne], w2b1, sem1).start()

        @pl.when(slot == 1)
        def _():
            pltpu.make_async_copy(w1_hbm.at[te_i], w1b1, sem1).wait()
            pltpu.make_async_copy(w2_hbm.at[te_i], w2b1, sem1).wait()

            @pl.when(hn == 1)
            def _():
                pltpu.make_async_copy(w1_hbm.at[ne], w1b0, sem0).start()
                pltpu.make_async_copy(w2_hbm.at[ne], w2b0, sem0).start()

    tv = meta_ref[i, 5]

    def compute(w1ref, w2ref):
        h = jnp.dot(xs_ref[...], w1ref[...], preferred_element_type=jnp.float32)
        h = h + b1_ref[0]
        h = jax.nn.gelu(h, approximate=True)
        y = jnp.dot(h, w2ref[...], preferred_element_type=jnp.float32)
        out_ref[...] = y + b2_ref[0]

    @pl.when((slot == 0) & (tv == 1))
    def _():
        compute(w1b0, w2b0)

    @pl.when((slot == 1) & (tv == 1))
    def _():
        compute(w1b1, w2b1)


def _grouped_mm(meta, xs, w1, b1, w2, b2):
    grid_spec = pltpu.PrefetchScalarGridSpec(
        num_scalar_prefetch=1,
        grid=(NTILES,),
        in_specs=[
            pl.BlockSpec((TILE, DD), lambda i, m: (i, 0)),
            pl.BlockSpec(memory_space=pltpu.MemorySpace.HBM),
            pl.BlockSpec((1, 1, II), lambda i, m: (m[i, 0], 0, 0)),
            pl.BlockSpec(memory_space=pltpu.MemorySpace.HBM),
            pl.BlockSpec((1, 1, DD), lambda i, m: (m[i, 0], 0, 0)),
        ],
        out_specs=pl.BlockSpec((TILE, DD), lambda i, m: (i, 0)),
        scratch_shapes=[
            pltpu.VMEM((DD, II), jnp.float32),
            pltpu.VMEM((DD, II), jnp.float32),
            pltpu.VMEM((II, DD), jnp.float32),
            pltpu.VMEM((II, DD), jnp.float32),
            pltpu.SemaphoreType.DMA,
            pltpu.SemaphoreType.DMA,
        ],
    )
    return pl.pallas_call(
        _mm_body,
        grid_spec=grid_spec,
        out_shape=jax.ShapeDtypeStruct((PROWS, DD), jnp.float32),
    )(meta, xs, w1, b1, w2, b2)


CCH = 32
NCH = TW // CCH


@functools.cache
def _combine_kernel():
    return functools.partial(
        pl.kernel,
        out_type=jax.ShapeDtypeStruct((NTOK, DD), jnp.float32),
        mesh=plsc.VectorSubcoreMesh(core_axis_name="c", subcore_axis_name="s"),
        scratch_types=[
            pltpu.VMEM((CCH, DD), jnp.float32),
            pltpu.VMEM((CCH, DD), jnp.float32),
            pltpu.VMEM((CCH, DD), jnp.float32),
            pltpu.VMEM((CCH, DD), jnp.float32),
            pltpu.VMEM((CCH,), jnp.int32),
            pltpu.VMEM((CCH,), jnp.int32),
            pltpu.VMEM((CCH,), jnp.int32),
            pltpu.VMEM((CCH,), jnp.int32),
            pltpu.VMEM((CCH, 16), jnp.float32),
            pltpu.VMEM((CCH, 16), jnp.float32),
            pltpu.VMEM((CCH, 16), jnp.float32),
            pltpu.VMEM((CCH, 16), jnp.float32),
            pltpu.SemaphoreType.DMA,
            pltpu.SemaphoreType.DMA,
            pltpu.SemaphoreType.DMA,
            pltpu.SemaphoreType.DMA,
        ],
    )(_combine_body)


def _combine_body(ys_hbm, pos0_hbm, pos1_hbm, w0_hbm, w1_hbm, out_hbm,
                  b0a, b1a, b0b, b1b, i0a, i1a, i0b, i1b,
                  w0a, w1a, w0b_, w1b_, sga, sgb, soa, sob):
    wid = lax.axis_index("s") * NC + lax.axis_index("c")
    base = wid * TW
    sets = [(b0a, b1a, i0a, i1a, w0a, w1a, sga, soa),
            (b0b, b1b, i0b, i1b, w0b_, w1b_, sgb, sob)]

    def issue(c):
        b0, b1, i0, i1, wv0, wv1, sg, so = sets[c % 2]
        cb = base + c * CCH
        pltpu.sync_copy(pos0_hbm.at[pl.ds(cb, CCH)], i0)
        pltpu.sync_copy(pos1_hbm.at[pl.ds(cb, CCH)], i1)
        pltpu.sync_copy(w0_hbm.at[pl.ds(cb, CCH)], wv0)
        pltpu.sync_copy(w1_hbm.at[pl.ds(cb, CCH)], wv1)
        pltpu.async_copy(ys_hbm.at[i0], b0, sg)
        pltpu.async_copy(ys_hbm.at[i1], b1, sg)

    def process(c):
        b0, b1, i0, i1, wv0, wv1, sg, so = sets[c % 2]
        cb = base + c * CCH
        pltpu.make_async_copy(ys_hbm.at[i0], b0, sg).wait()
        pltpu.make_async_copy(ys_hbm.at[i1], b1, sg).wait()

        def row_body(i, carry):
            w0row = wv0[i, :]
            w1row = wv1[i, :]
            for j in range(DD // 16):
                s = pl.ds(j * 16, 16)
                b0[i, s] = w0row * b0[i, s] + w1row * b1[i, s]
            return carry

        lax.fori_loop(0, CCH, row_body, 0)
        pltpu.async_copy(b0, out_hbm.at[pl.ds(cb, CCH)], so)

    issue(0)
    for c in range(NCH):
        if c + 1 < NCH:
            if c + 1 >= 2:
                b0n, _, _, _, _, _, _, son = sets[(c + 1) % 2]
                pltpu.make_async_copy(b0n, out_hbm.at[pl.ds(base, CCH)],
                                      son).wait()
            issue(c + 1)
        process(c)
    b0l, _, _, _, _, _, _, sol = sets[(NCH - 1) % 2]
    pltpu.make_async_copy(b0l, out_hbm.at[pl.ds(base, CCH)], sol).wait()
    b0p, _, _, _, _, _, _, sop = sets[(NCH - 2) % 2]
    pltpu.make_async_copy(b0p, out_hbm.at[pl.ds(base, CCH)], sop).wait()


def kernel(hidden_states, style_emb, Wg, W1, b1, W2, b2):
    x = hidden_states.reshape(NTOK, DD)
    style8 = jnp.zeros((8, DD), jnp.float32).at[:BB].set(style_emb)
    pos0, pos1, w0b, w1b, meta, usage = _router(x, style8, Wg)
    pos0 = pos0.reshape(NTOK)
    pos1 = pos1.reshape(NTOK)
    xs = _dispatch_kernel()(x, pos0, pos1)
    ys = _grouped_mm(meta, xs, W1,
                     b1.reshape(EE, 1, II), W2, b2.reshape(EE, 1, DD))
    out = _combine_kernel()(ys, pos0, pos1, w0b, w1b)
    return out.reshape(BB, LL, DD), usage.reshape(EE)
